# baseline (speedup 1.0000x reference)
import jax
import jax.numpy as jnp
from jax.experimental import pallas as pl
from jax.experimental.pallas import tpu as pltpu

_BETA = 0.25
_N_TOK = 8192
_D = 32
_ROWS = 16384
_TILE = 512
_STEPS = _ROWS // _TILE


def _row_norms(sq):
    acc = ((sq[:, 0:8] + sq[:, 8:16]) + sq[:, 16:24]) + sq[:, 24:32]
    t = acc[:, 0:4] + acc[:, 4:8]
    t = t[:, 0:2] + t[:, 2:4]
    return t[:, 0:1] + t[:, 1:2]


def _col_norms(sq):
    acc = ((sq[0:8, :] + sq[8:16, :]) + sq[16:24, :]) + sq[24:32, :]
    t = acc[0:4, :] + acc[4:8, :]
    t = t[0:2, :] + t[2:4, :]
    return t[0:1, :] + t[1:2, :]


def _vq_kernel(z_ref, w_ref, wt_ref, zq_ref, idx_ref, counts_ref, loss_ref,
               perp_ref, w2_ref):
    step = pl.program_id(0)
    z = z_ref[...]
    w = w_ref[...]

    @pl.when(step == 0)
    def _w2_once():
        wt = wt_ref[...]
        w2_ref[...] = _col_norms(wt * wt)

    z2 = _row_norms(z * z)
    w2 = w2_ref[...]
    wneg2 = w.astype(jnp.bfloat16) * jnp.bfloat16(-2.0)
    prod = jax.lax.dot_general(
        z.astype(jnp.bfloat16), wneg2,
        (((1,), (1,)), ((), ())),
        preferred_element_type=jnp.float32)
    d = (z2 + w2) + prod

    dmin = jnp.min(d, axis=1, keepdims=True)
    iota = jax.lax.broadcasted_iota(jnp.int32, d.shape, 1)
    idx = jnp.min(jnp.where(d == dmin, iota, _N_TOK), axis=1)

    onehot = (iota == idx[:, None]).astype(jnp.float32)
    zq = jax.lax.dot_general(
        onehot, w, (((1,), (0,)), ((), ())),
        preferred_element_type=jnp.float32)

    diff = zq - z
    zq_ref[...] = z + diff
    idx_ref[...] = idx.reshape(1, 1, _TILE)

    ones_row = jnp.full((1, _TILE), 1.0, jnp.float32)
    part_counts = jax.lax.dot_general(
        ones_row, onehot, (((1,), (0,)), ((), ())),
        preferred_element_type=jnp.float32)
    part_loss = jnp.sum(diff * diff)

    @pl.when(step == 0)
    def _init():
        counts_ref[...] = part_counts
        loss_ref[0, 0] = part_loss

    @pl.when(step > 0)
    def _accum():
        counts_ref[...] += part_counts
        loss_ref[0, 0] += part_loss

    @pl.when(step == _STEPS - 1)
    def _finish():
        loss_ref[0, 0] = loss_ref[0, 0] * (_BETA / (_ROWS * _D))
        e_mean = counts_ref[...] * (1.0 / _ROWS)
        perp_ref[0, 0] = jnp.exp(-jnp.sum(e_mean * jnp.log(e_mean + 1e-10)))


def kernel(z, weight):
    z_flat = z.reshape(_ROWS, _D)
    zq_st, idx3, _counts, loss, perp = pl.pallas_call(
        _vq_kernel,
        grid=(_STEPS,),
        in_specs=[
            pl.BlockSpec((_TILE, _D), lambda i: (i, 0)),
            pl.BlockSpec((_N_TOK, _D), lambda i: (0, 0)),
            pl.BlockSpec((_D, _N_TOK), lambda i: (0, 0)),
        ],
        out_specs=[
            pl.BlockSpec((_TILE, _D), lambda i: (i, 0)),
            pl.BlockSpec((1, 1, _TILE), lambda i: (i, 0, 0)),
            pl.BlockSpec((1, _N_TOK), lambda i: (0, 0)),
            pl.BlockSpec(memory_space=pltpu.SMEM),
            pl.BlockSpec(memory_space=pltpu.SMEM),
        ],
        out_shape=[
            jax.ShapeDtypeStruct((_ROWS, _D), jnp.float32),
            jax.ShapeDtypeStruct((_STEPS, 1, _TILE), jnp.int32),
            jax.ShapeDtypeStruct((1, _N_TOK), jnp.float32),
            jax.ShapeDtypeStruct((1, 1), jnp.float32),
            jax.ShapeDtypeStruct((1, 1), jnp.float32),
        ],
        scratch_shapes=[pltpu.VMEM((1, _N_TOK), jnp.float32)],
        compiler_params=pltpu.CompilerParams(
            dimension_semantics=("arbitrary",),
        ),
    )(z_flat, weight, weight.T)
    return (loss[0, 0], zq_st.reshape(z.shape), idx3.reshape(_ROWS), perp[0, 0])


# speedup vs baseline: 1.0129x; 1.0129x over previous
import jax
import jax.numpy as jnp
from jax.experimental import pallas as pl
from jax.experimental.pallas import tpu as pltpu

_BETA = 0.25
_N_TOK = 8192
_D = 32
_ROWS = 16384
_TILE = 512
_STEPS = _ROWS // _TILE


def _row_norms(sq):
    acc = ((sq[:, 0:8] + sq[:, 8:16]) + sq[:, 16:24]) + sq[:, 24:32]
    t = acc[:, 0:4] + acc[:, 4:8]
    t = t[:, 0:2] + t[:, 2:4]
    return t[:, 0:1] + t[:, 1:2]


def _col_norms(sq):
    acc = ((sq[0:8, :] + sq[8:16, :]) + sq[16:24, :]) + sq[24:32, :]
    t = acc[0:4, :] + acc[4:8, :]
    t = t[0:2, :] + t[2:4, :]
    return t[0:1, :] + t[1:2, :]


def _vq_kernel(z_ref, w_ref, wt_ref, zq_ref, idx_ref, counts_ref, loss_ref,
               perp_ref, w2_ref):
    step = pl.program_id(0)
    z = z_ref[...]
    w = w_ref[...]

    @pl.when(step == 0)
    def _w2_once():
        wt = wt_ref[...]
        w2_ref[...] = _col_norms(wt * wt)

    z2 = _row_norms(z * z)
    w2 = w2_ref[...]
    prod = jax.lax.dot_general(
        z.astype(jnp.bfloat16), w.astype(jnp.bfloat16),
        (((1,), (1,)), ((), ())),
        preferred_element_type=jnp.float32)
    d = (z2 + w2) - 2.0 * prod

    dmin = jnp.min(d, axis=1, keepdims=True)
    iota = jax.lax.broadcasted_iota(jnp.int32, d.shape, 1)
    idx = jnp.min(jnp.where(d == dmin, iota, _N_TOK), axis=1)

    onehot = (iota == idx[:, None]).astype(jnp.float32)
    zq = jax.lax.dot_general(
        onehot, w, (((1,), (0,)), ((), ())),
        preferred_element_type=jnp.float32)

    diff = zq - z
    zq_ref[...] = z + diff
    idx_ref[...] = idx.reshape(1, 1, _TILE)

    ones_row = jnp.full((1, _TILE), 1.0, jnp.float32)
    part_counts = jax.lax.dot_general(
        ones_row, onehot, (((1,), (0,)), ((), ())),
        preferred_element_type=jnp.float32)
    part_loss = jnp.sum(diff * diff)

    @pl.when(step == 0)
    def _init():
        counts_ref[...] = part_counts
        loss_ref[0, 0] = part_loss

    @pl.when(step > 0)
    def _accum():
        counts_ref[...] += part_counts
        loss_ref[0, 0] += part_loss

    @pl.when(step == _STEPS - 1)
    def _finish():
        loss_ref[0, 0] = loss_ref[0, 0] * (_BETA / (_ROWS * _D))
        e_mean = counts_ref[...] * (1.0 / _ROWS)
        perp_ref[0, 0] = jnp.exp(-jnp.sum(e_mean * jnp.log(e_mean + 1e-10)))


def kernel(z, weight):
    z_flat = z.reshape(_ROWS, _D)
    zq_st, idx3, _counts, loss, perp = pl.pallas_call(
        _vq_kernel,
        grid=(_STEPS,),
        in_specs=[
            pl.BlockSpec((_TILE, _D), lambda i: (i, 0)),
            pl.BlockSpec((_N_TOK, _D), lambda i: (0, 0)),
            pl.BlockSpec((_D, _N_TOK), lambda i: (0, 0)),
        ],
        out_specs=[
            pl.BlockSpec((_TILE, _D), lambda i: (i, 0)),
            pl.BlockSpec((1, 1, _TILE), lambda i: (i, 0, 0)),
            pl.BlockSpec((1, _N_TOK), lambda i: (0, 0)),
            pl.BlockSpec(memory_space=pltpu.SMEM),
            pl.BlockSpec(memory_space=pltpu.SMEM),
        ],
        out_shape=[
            jax.ShapeDtypeStruct((_ROWS, _D), jnp.float32),
            jax.ShapeDtypeStruct((_STEPS, 1, _TILE), jnp.int32),
            jax.ShapeDtypeStruct((1, _N_TOK), jnp.float32),
            jax.ShapeDtypeStruct((1, 1), jnp.float32),
            jax.ShapeDtypeStruct((1, 1), jnp.float32),
        ],
        scratch_shapes=[pltpu.VMEM((1, _N_TOK), jnp.float32)],
        compiler_params=pltpu.CompilerParams(
            dimension_semantics=("arbitrary",),
        ),
    )(z_flat, weight, weight.T)
    return (loss[0, 0], zq_st.reshape(z.shape), idx3.reshape(_ROWS), perp[0, 0])
